# baseline (speedup 1.0000x reference)
import functools

import jax
import jax.numpy as jnp
from jax import lax
from jax.experimental import pallas as pl
from jax.experimental.pallas import tpu as pltpu
from jax.experimental.pallas import tpu_sc as plsc

NC = 2
NS = 16
LANES = 16
CHUNK = 128
SEC = 16


def _matmul_body(x_ref, w_ref, o_ref):
    o_ref[0] = jnp.dot(x_ref[...], w_ref[0],
                       preferred_element_type=jnp.float32)


def _combine_body(p_ref, b_ref, o_ref):
    o_ref[...] = jnp.concatenate([p_ref[0], p_ref[1]], axis=1) + b_ref[...]


def _make_sc_spmm(n_vpad, c_half, n_chunks):
    width = 2 * c_half
    packrows = n_vpad // 2
    rows_per_tile = packrows // NS
    assert n_chunks % SEC == 0
    nsec = n_chunks // SEC
    mesh = plsc.VectorSubcoreMesh(core_axis_name="c", subcore_axis_name="s",
                                  num_cores=NC, num_subcores=NS)

    @functools.partial(
        pl.kernel,
        out_type=jax.ShapeDtypeStruct((NC, packrows, width), jnp.float32),
        mesh=mesh,
        scratch_types=[
            pltpu.VMEM((SEC, CHUNK), jnp.int32),
            pltpu.VMEM((SEC, CHUNK), jnp.int32),
            pltpu.VMEM((SEC, CHUNK), jnp.float32),
            pltpu.VMEM((SEC, CHUNK), jnp.int32),
            pltpu.VMEM((CHUNK, width), jnp.float32),
            pltpu.VMEM((CHUNK, width), jnp.float32),
            pltpu.VMEM_SHARED((packrows, width), jnp.float32),
            pltpu.VMEM_SHARED((packrows, width), jnp.float32),
            pltpu.SemaphoreType.DMA,
            pltpu.SemaphoreType.DMA,
            pltpu.SemaphoreType.DMA,
            pltpu.SemaphoreType.DMA,
        ],
    )
    def sc_spmm(xw_hbm, col_hbm, row_hbm, val_hbm, par_hbm, zeros_hbm,
                out_hbm, col_all, row_all, val_all, par_all, gat0, gat1,
                xw_sh, acc_sh, semg0, semg1, sems0, sems1):
        c = lax.axis_index("c")
        s = lax.axis_index("s")
        rows = pl.ds(s * rows_per_tile, rows_per_tile)

        pltpu.sync_copy(xw_hbm.at[c, rows, :], xw_sh.at[rows, :])
        pltpu.sync_copy(zeros_hbm, acc_sh.at[rows, :])
        plsc.subcore_barrier()

        gats = (gat0, gat1)
        semg = (semg0, semg1)
        sems = (sems0, sems1)
        zero16 = jnp.zeros((LANES,), jnp.float32)
        ngrp = c_half // LANES

        @pl.loop(0, nsec)
        def section(h):
            sec0 = h * SEC
            pltpu.sync_copy(col_hbm.at[s, pl.ds(sec0, SEC)], col_all)
            pltpu.sync_copy(row_hbm.at[s, pl.ds(sec0, SEC)], row_all)
            pltpu.sync_copy(val_hbm.at[s, pl.ds(sec0, SEC)], val_all)
            pltpu.sync_copy(par_hbm.at[s, pl.ds(sec0, SEC)], par_all)

            pltpu.async_copy(xw_sh.at[col_all.at[0]], gat0, semg0)
            pltpu.async_copy(xw_sh.at[col_all.at[1]], gat1, semg1)

            @pl.loop(0, SEC, step=2)
            def chunk_body(k):
                for b in range(2):
                    kk = k + b
                    gat = gats[b]

                    pltpu.make_async_copy(
                        xw_sh.at[col_all.at[kk]], gat, semg[b]).wait()

                    @plsc.parallel_loop(0, CHUNK // LANES, unroll=2)
                    def scale_body(t):
                        v16 = val_all[kk, pl.ds(t * LANES, LANES)]
                        p16 = par_all[kk, pl.ds(t * LANES, LANES)]
                        for i in range(LANES):
                            e = t * LANES + i
                            par_i = p16[i]
                            pcf = (par_i & 1).astype(jnp.float32)
                            prf = (par_i >> 1).astype(jnp.float32)
                            pc1 = jnp.broadcast_to(pcf, (LANES,))
                            pc0 = jnp.broadcast_to(1.0 - pcf, (LANES,))
                            pr1 = jnp.broadcast_to(prf, (LANES,))
                            pr0 = jnp.broadcast_to(1.0 - prf, (LANES,))
                            scale = jnp.broadcast_to(v16[i], (LANES,))
                            for g in range(ngrp):
                                lo = pl.ds(g * LANES, LANES)
                                hi = pl.ds(c_half + g * LANES, LANES)
                                a = gat[e, lo]
                                bb = gat[e, hi]
                                m = (a * pc0 + bb * pc1) * scale
                                gat[e, lo] = m * pr0
                                gat[e, hi] = m * pr1

                    pltpu.async_copy(gat, acc_sh.at[row_all.at[kk]],
                                     sems[b], add=True)

                for b in range(2):
                    kk = k + b
                    gat = gats[b]

                    @pl.when(kk + 2 < SEC)
                    def _():
                        pltpu.make_async_copy(
                            gat, acc_sh.at[row_all.at[kk]], sems[b]).wait()
                        pltpu.async_copy(
                            xw_sh.at[col_all.at[kk + 2]], gat, semg[b])

            pltpu.make_async_copy(
                gat0, acc_sh.at[row_all.at[SEC - 2]], sems0).wait()
            pltpu.make_async_copy(
                gat1, acc_sh.at[row_all.at[SEC - 1]], sems1).wait()

        plsc.subcore_barrier()

        pltpu.sync_copy(acc_sh.at[rows, :], out_hbm.at[c, rows, :])

    return sc_spmm


def kernel(x, edge_index, edge_vals, W, b):
    _, n_vertex, c_in = x.shape
    c_out = W.shape[1]
    c_half = c_out // NC
    n_edges = edge_vals.shape[0]
    n_vpad = -(-n_vertex // (16 * NS)) * (16 * NS)

    x2d = jnp.zeros((n_vpad, c_in), jnp.float32).at[:n_vertex].set(
        x.reshape(n_vertex, c_in))
    row_blk = n_vpad // NS
    W3 = jnp.stack([W[:, c * c_half:(c + 1) * c_half] for c in range(NC)])
    xw = pl.pallas_call(
        _matmul_body,
        grid=(NS, NC),
        in_specs=[
            pl.BlockSpec((row_blk, c_in), lambda i, j: (i, 0)),
            pl.BlockSpec((1, c_in, c_half), lambda i, j: (j, 0, 0)),
        ],
        out_specs=pl.BlockSpec((1, row_blk, c_half), lambda i, j: (j, i, 0)),
        out_shape=jax.ShapeDtypeStruct((NC, n_vpad, c_half), jnp.float32),
    )(x2d, W3)
    xw_pack = xw.reshape(NC, n_vpad // 2, c_out)

    per_tile = -(-n_edges // NS)
    n_chunks = -(-per_tile // CHUNK)
    n_chunks += -n_chunks % SEC
    e_pad = NS * n_chunks * CHUNK
    pad = e_pad - n_edges
    col = jnp.concatenate([edge_index[1], jnp.zeros((pad,), jnp.int32)])
    row = jnp.concatenate([edge_index[0], jnp.zeros((pad,), jnp.int32)])
    val = jnp.concatenate([edge_vals, jnp.zeros((pad,), jnp.float32)])
    par = (col & 1) | ((row & 1) << 1)
    colh3 = (col >> 1).reshape(NS, n_chunks, CHUNK)
    rowh3 = (row >> 1).reshape(NS, n_chunks, CHUNK)
    val3 = val.reshape(NS, n_chunks, CHUNK)
    par3 = par.reshape(NS, n_chunks, CHUNK)
    zeros = jnp.zeros((n_vpad // 2 // NS, c_out), jnp.float32)
    parts = _make_sc_spmm(n_vpad, c_half, n_chunks)(
        xw_pack, colh3, rowh3, val3, par3, zeros)
    parts = parts.reshape(NC, n_vpad, c_half)

    out_blk = 1000
    grid_m = n_vertex // out_blk
    out = pl.pallas_call(
        _combine_body,
        grid=(grid_m,),
        in_specs=[
            pl.BlockSpec((NC, out_blk, c_half), lambda i: (0, i, 0)),
            pl.BlockSpec((1, c_out), lambda i: (0, 0)),
        ],
        out_specs=pl.BlockSpec((out_blk, c_out), lambda i: (i, 0)),
        out_shape=jax.ShapeDtypeStruct((n_vertex, c_out), jnp.float32),
    )(parts, b.reshape(1, c_out))
    return out


# speedup vs baseline: 1.1408x; 1.1408x over previous
import functools

import jax
import jax.numpy as jnp
from jax import lax
from jax.experimental import pallas as pl
from jax.experimental.pallas import tpu as pltpu
from jax.experimental.pallas import tpu_sc as plsc

NC = 2
NS = 16
LANES = 16
CHUNK = 128
SEC = 16


def _matmul_body(x_ref, w_ref, o_ref):
    o_ref[0] = jnp.dot(x_ref[...], w_ref[0],
                       preferred_element_type=jnp.float32)


def _combine_body(p_ref, b_ref, o_ref):
    o_ref[...] = jnp.concatenate([p_ref[0], p_ref[1]], axis=1) + b_ref[...]


def _make_sc_spmm(n_vpad, c_half, n_chunks):
    width = 2 * c_half
    packrows = n_vpad // 2
    rows_per_tile = packrows // NS
    assert n_chunks % SEC == 0
    nsec = n_chunks // SEC
    mesh = plsc.VectorSubcoreMesh(core_axis_name="c", subcore_axis_name="s",
                                  num_cores=NC, num_subcores=NS)

    @functools.partial(
        pl.kernel,
        out_type=jax.ShapeDtypeStruct((NC, packrows, width), jnp.float32),
        mesh=mesh,
        scratch_types=[
            pltpu.VMEM((SEC, CHUNK), jnp.int32),
            pltpu.VMEM((SEC, CHUNK), jnp.int32),
            pltpu.VMEM((SEC, CHUNK), jnp.float32),
            pltpu.VMEM((SEC, CHUNK), jnp.int32),
            pltpu.VMEM((CHUNK, width), jnp.float32),
            pltpu.VMEM((CHUNK, width), jnp.float32),
            pltpu.VMEM_SHARED((packrows, width), jnp.float32),
            pltpu.VMEM_SHARED((packrows, width), jnp.float32),
            pltpu.SemaphoreType.DMA,
            pltpu.SemaphoreType.DMA,
            pltpu.SemaphoreType.DMA,
            pltpu.SemaphoreType.DMA,
        ],
    )
    def sc_spmm(xw_hbm, col_hbm, row_hbm, val_hbm, par_hbm, zeros_hbm,
                out_hbm, col_all, row_all, val_all, par_all, gat0, gat1,
                xw_sh, acc_sh, semg0, semg1, sems0, sems1):
        c = lax.axis_index("c")
        s = lax.axis_index("s")
        rows = pl.ds(s * rows_per_tile, rows_per_tile)

        pltpu.sync_copy(xw_hbm.at[c, rows, :], xw_sh.at[rows, :])
        pltpu.sync_copy(zeros_hbm, acc_sh.at[rows, :])
        plsc.subcore_barrier()

        gats = (gat0, gat1)
        semg = (semg0, semg1)
        sems = (sems0, sems1)
        zero16 = jnp.zeros((LANES,), jnp.float32)
        ngrp = c_half // LANES

        @pl.loop(0, nsec)
        def section(h):
            sec0 = h * SEC
            pltpu.sync_copy(col_hbm.at[s, pl.ds(sec0, SEC)], col_all)
            pltpu.sync_copy(row_hbm.at[s, pl.ds(sec0, SEC)], row_all)
            pltpu.sync_copy(val_hbm.at[s, pl.ds(sec0, SEC)], val_all)
            pltpu.sync_copy(par_hbm.at[s, pl.ds(sec0, SEC)], par_all)

            pltpu.async_copy(xw_sh.at[col_all.at[0]], gat0, semg0)
            pltpu.async_copy(xw_sh.at[col_all.at[1]], gat1, semg1)

            @pl.loop(0, SEC, step=2)
            def chunk_body(k):
                for b in range(2):
                    kk = k + b
                    gat = gats[b]

                    pltpu.make_async_copy(
                        xw_sh.at[col_all.at[kk]], gat, semg[b]).wait()

                    @plsc.parallel_loop(0, CHUNK // LANES)
                    def scale_body(t):
                        v16 = val_all[kk, pl.ds(t * LANES, LANES)]
                        p16 = par_all[kk, pl.ds(t * LANES, LANES)]
                        for i in range(LANES):
                            e = t * LANES + i
                            par_i = p16[i]
                            pcf = (par_i & 1).astype(jnp.float32)
                            prf = (par_i >> 1).astype(jnp.float32)
                            pc1 = jnp.broadcast_to(pcf, (LANES,))
                            pc0 = jnp.broadcast_to(1.0 - pcf, (LANES,))
                            pr1 = jnp.broadcast_to(prf, (LANES,))
                            pr0 = jnp.broadcast_to(1.0 - prf, (LANES,))
                            scale = jnp.broadcast_to(v16[i], (LANES,))
                            for g in range(ngrp):
                                lo = pl.ds(g * LANES, LANES)
                                hi = pl.ds(c_half + g * LANES, LANES)
                                a = gat[e, lo]
                                bb = gat[e, hi]
                                m = (a * pc0 + bb * pc1) * scale
                                gat[e, lo] = m * pr0
                                gat[e, hi] = m * pr1

                    pltpu.async_copy(gat, acc_sh.at[row_all.at[kk]],
                                     sems[b], add=True)

                for b in range(2):
                    kk = k + b
                    gat = gats[b]

                    @pl.when(kk + 2 < SEC)
                    def _():
                        pltpu.make_async_copy(
                            gat, acc_sh.at[row_all.at[kk]], sems[b]).wait()
                        pltpu.async_copy(
                            xw_sh.at[col_all.at[kk + 2]], gat, semg[b])

            pltpu.make_async_copy(
                gat0, acc_sh.at[row_all.at[SEC - 2]], sems0).wait()
            pltpu.make_async_copy(
                gat1, acc_sh.at[row_all.at[SEC - 1]], sems1).wait()

        plsc.subcore_barrier()

        pltpu.sync_copy(acc_sh.at[rows, :], out_hbm.at[c, rows, :])

    return sc_spmm


def kernel(x, edge_index, edge_vals, W, b):
    _, n_vertex, c_in = x.shape
    c_out = W.shape[1]
    c_half = c_out // NC
    n_edges = edge_vals.shape[0]
    n_vpad = -(-n_vertex // (16 * NS)) * (16 * NS)

    x2d = jnp.zeros((n_vpad, c_in), jnp.float32).at[:n_vertex].set(
        x.reshape(n_vertex, c_in))
    row_blk = n_vpad // NS
    W3 = jnp.stack([W[:, c * c_half:(c + 1) * c_half] for c in range(NC)])
    xw = pl.pallas_call(
        _matmul_body,
        grid=(NS, NC),
        in_specs=[
            pl.BlockSpec((row_blk, c_in), lambda i, j: (i, 0)),
            pl.BlockSpec((1, c_in, c_half), lambda i, j: (j, 0, 0)),
        ],
        out_specs=pl.BlockSpec((1, row_blk, c_half), lambda i, j: (j, i, 0)),
        out_shape=jax.ShapeDtypeStruct((NC, n_vpad, c_half), jnp.float32),
    )(x2d, W3)
    xw_pack = xw.reshape(NC, n_vpad // 2, c_out)

    per_tile = -(-n_edges // NS)
    n_chunks = -(-per_tile // CHUNK)
    n_chunks += -n_chunks % SEC
    e_pad = NS * n_chunks * CHUNK
    pad = e_pad - n_edges
    col = jnp.concatenate([edge_index[1], jnp.zeros((pad,), jnp.int32)])
    row = jnp.concatenate([edge_index[0], jnp.zeros((pad,), jnp.int32)])
    val = jnp.concatenate([edge_vals, jnp.zeros((pad,), jnp.float32)])
    par = (col & 1) | ((row & 1) << 1)
    colh3 = (col >> 1).reshape(NS, n_chunks, CHUNK)
    rowh3 = (row >> 1).reshape(NS, n_chunks, CHUNK)
    val3 = val.reshape(NS, n_chunks, CHUNK)
    par3 = par.reshape(NS, n_chunks, CHUNK)
    zeros = jnp.zeros((n_vpad // 2 // NS, c_out), jnp.float32)
    parts = _make_sc_spmm(n_vpad, c_half, n_chunks)(
        xw_pack, colh3, rowh3, val3, par3, zeros)
    parts = parts.reshape(NC, n_vpad, c_half)

    out_blk = 1000
    grid_m = n_vertex // out_blk
    out = pl.pallas_call(
        _combine_body,
        grid=(grid_m,),
        in_specs=[
            pl.BlockSpec((NC, out_blk, c_half), lambda i: (0, i, 0)),
            pl.BlockSpec((1, c_out), lambda i: (0, 0)),
        ],
        out_specs=pl.BlockSpec((out_blk, c_out), lambda i: (i, 0)),
        out_shape=jax.ShapeDtypeStruct((n_vertex, c_out), jnp.float32),
    )(parts, b.reshape(1, c_out))
    return out


# speedup vs baseline: 1.2409x; 1.0877x over previous
import functools

import jax
import jax.numpy as jnp
from jax import lax
from jax.experimental import pallas as pl
from jax.experimental.pallas import tpu as pltpu
from jax.experimental.pallas import tpu_sc as plsc

NC = 2
NS = 16
LANES = 16
CHUNK = 128
SEC = 16


def _matmul_body(x_ref, w_ref, o_ref):
    o_ref[0] = jnp.dot(x_ref[...], w_ref[0],
                       preferred_element_type=jnp.float32)


def _combine_body(p_ref, b_ref, o_ref):
    o_ref[...] = jnp.concatenate([p_ref[0], p_ref[1]], axis=1) + b_ref[...]


def _make_sc_spmm(n_vpad, c_half, n_chunks):
    width = 2 * c_half
    packrows = n_vpad // 2
    rows_per_tile = packrows // NS
    assert n_chunks % SEC == 0
    nsec = n_chunks // SEC
    mesh = plsc.VectorSubcoreMesh(core_axis_name="c", subcore_axis_name="s",
                                  num_cores=NC, num_subcores=NS)

    @functools.partial(
        pl.kernel,
        out_type=jax.ShapeDtypeStruct((NC, packrows, width), jnp.float32),
        mesh=mesh,
        scratch_types=[
            pltpu.VMEM((SEC, CHUNK), jnp.int32),
            pltpu.VMEM((SEC, CHUNK), jnp.int32),
            pltpu.VMEM((SEC, CHUNK), jnp.float32),
            pltpu.VMEM((SEC, CHUNK), jnp.int32),
            pltpu.VMEM((CHUNK, width), jnp.float32),
            pltpu.VMEM((CHUNK, width), jnp.float32),
            pltpu.VMEM_SHARED((packrows, width), jnp.float32),
            pltpu.VMEM_SHARED((packrows, width), jnp.float32),
            pltpu.SemaphoreType.DMA,
            pltpu.SemaphoreType.DMA,
            pltpu.SemaphoreType.DMA,
            pltpu.SemaphoreType.DMA,
        ],
    )
    def sc_spmm(xw_hbm, col_hbm, row_hbm, val_hbm, par_hbm, zeros_hbm,
                out_hbm, col_all, row_all, val_all, par_all, gat0, gat1,
                xw_sh, acc_sh, semg0, semg1, sems0, sems1):
        c = lax.axis_index("c")
        s = lax.axis_index("s")
        rows = pl.ds(s * rows_per_tile, rows_per_tile)

        pltpu.sync_copy(xw_hbm.at[c, rows, :], xw_sh.at[rows, :])
        pltpu.sync_copy(zeros_hbm, acc_sh.at[rows, :])
        plsc.subcore_barrier()

        gats = (gat0, gat1)
        semg = (semg0, semg1)
        sems = (sems0, sems1)
        zero16 = jnp.zeros((LANES,), jnp.float32)
        ngrp = c_half // LANES

        @pl.loop(0, nsec)
        def section(h):
            sec0 = h * SEC
            pltpu.async_copy(col_hbm.at[s, pl.ds(sec0, SEC)], col_all, sems0)
            pltpu.async_copy(row_hbm.at[s, pl.ds(sec0, SEC)], row_all, sems0)
            pltpu.async_copy(val_hbm.at[s, pl.ds(sec0, SEC)], val_all, sems0)
            pltpu.async_copy(par_hbm.at[s, pl.ds(sec0, SEC)], par_all, sems0)
            pltpu.make_async_copy(col_hbm.at[s, pl.ds(sec0, SEC)], col_all,
                                  sems0).wait()
            pltpu.make_async_copy(row_hbm.at[s, pl.ds(sec0, SEC)], row_all,
                                  sems0).wait()
            pltpu.make_async_copy(val_hbm.at[s, pl.ds(sec0, SEC)], val_all,
                                  sems0).wait()
            pltpu.make_async_copy(par_hbm.at[s, pl.ds(sec0, SEC)], par_all,
                                  sems0).wait()

            pltpu.async_copy(xw_sh.at[col_all.at[0]], gat0, semg0)
            pltpu.async_copy(xw_sh.at[col_all.at[1]], gat1, semg1)

            @pl.loop(0, SEC, step=2)
            def chunk_body(k):
                for b in range(2):
                    kk = k + b
                    gat = gats[b]

                    pltpu.make_async_copy(
                        xw_sh.at[col_all.at[kk]], gat, semg[b]).wait()

                    @plsc.parallel_loop(0, CHUNK // LANES)
                    def scale_body(t):
                        v16 = val_all[kk, pl.ds(t * LANES, LANES)]
                        p16 = par_all[kk, pl.ds(t * LANES, LANES)]
                        for i in range(LANES):
                            e = t * LANES + i
                            par_i = p16[i]
                            pcf = (par_i & 1).astype(jnp.float32)
                            prf = (par_i >> 1).astype(jnp.float32)
                            pc1 = jnp.broadcast_to(pcf, (LANES,))
                            pc0 = jnp.broadcast_to(1.0 - pcf, (LANES,))
                            pr1 = jnp.broadcast_to(prf, (LANES,))
                            pr0 = jnp.broadcast_to(1.0 - prf, (LANES,))
                            scale = jnp.broadcast_to(v16[i], (LANES,))
                            for g in range(ngrp):
                                lo = pl.ds(g * LANES, LANES)
                                hi = pl.ds(c_half + g * LANES, LANES)
                                a = gat[e, lo]
                                bb = gat[e, hi]
                                m = (a * pc0 + bb * pc1) * scale
                                gat[e, lo] = m * pr0
                                gat[e, hi] = m * pr1

                    pltpu.async_copy(gat, acc_sh.at[row_all.at[kk]],
                                     sems[b], add=True)

                for b in range(2):
                    kk = k + b
                    gat = gats[b]

                    @pl.when(kk + 2 < SEC)
                    def _():
                        pltpu.make_async_copy(
                            gat, acc_sh.at[row_all.at[kk]], sems[b]).wait()
                        pltpu.async_copy(
                            xw_sh.at[col_all.at[kk + 2]], gat, semg[b])

            pltpu.make_async_copy(
                gat0, acc_sh.at[row_all.at[SEC - 2]], sems0).wait()
            pltpu.make_async_copy(
                gat1, acc_sh.at[row_all.at[SEC - 1]], sems1).wait()

        plsc.subcore_barrier()

        pltpu.sync_copy(acc_sh.at[rows, :], out_hbm.at[c, rows, :])

    return sc_spmm


def kernel(x, edge_index, edge_vals, W, b):
    _, n_vertex, c_in = x.shape
    c_out = W.shape[1]
    c_half = c_out // NC
    n_edges = edge_vals.shape[0]
    n_vpad = -(-n_vertex // (16 * NS)) * (16 * NS)

    x2d = jnp.zeros((n_vpad, c_in), jnp.float32).at[:n_vertex].set(
        x.reshape(n_vertex, c_in))
    row_blk = n_vpad // NS
    W3 = jnp.stack([W[:, c * c_half:(c + 1) * c_half] for c in range(NC)])
    xw = pl.pallas_call(
        _matmul_body,
        grid=(NS, NC),
        in_specs=[
            pl.BlockSpec((row_blk, c_in), lambda i, j: (i, 0)),
            pl.BlockSpec((1, c_in, c_half), lambda i, j: (j, 0, 0)),
        ],
        out_specs=pl.BlockSpec((1, row_blk, c_half), lambda i, j: (j, i, 0)),
        out_shape=jax.ShapeDtypeStruct((NC, n_vpad, c_half), jnp.float32),
    )(x2d, W3)
    xw_pack = xw.reshape(NC, n_vpad // 2, c_out)

    per_tile = -(-n_edges // NS)
    n_chunks = -(-per_tile // CHUNK)
    n_chunks += -n_chunks % SEC
    e_pad = NS * n_chunks * CHUNK
    pad = e_pad - n_edges
    col = jnp.concatenate([edge_index[1], jnp.zeros((pad,), jnp.int32)])
    row = jnp.concatenate([edge_index[0], jnp.zeros((pad,), jnp.int32)])
    val = jnp.concatenate([edge_vals, jnp.zeros((pad,), jnp.float32)])
    par = (col & 1) | ((row & 1) << 1)
    colh3 = (col >> 1).reshape(NS, n_chunks, CHUNK)
    rowh3 = (row >> 1).reshape(NS, n_chunks, CHUNK)
    val3 = val.reshape(NS, n_chunks, CHUNK)
    par3 = par.reshape(NS, n_chunks, CHUNK)
    zeros = jnp.zeros((n_vpad // 2 // NS, c_out), jnp.float32)
    parts = _make_sc_spmm(n_vpad, c_half, n_chunks)(
        xw_pack, colh3, rowh3, val3, par3, zeros)
    parts = parts.reshape(NC, n_vpad, c_half)

    out_blk = 1000
    grid_m = n_vertex // out_blk
    out = pl.pallas_call(
        _combine_body,
        grid=(grid_m,),
        in_specs=[
            pl.BlockSpec((NC, out_blk, c_half), lambda i: (0, i, 0)),
            pl.BlockSpec((1, c_out), lambda i: (0, 0)),
        ],
        out_specs=pl.BlockSpec((out_blk, c_out), lambda i: (i, 0)),
        out_shape=jax.ShapeDtypeStruct((n_vertex, c_out), jnp.float32),
    )(parts, b.reshape(1, c_out))
    return out
